# baseline (speedup 1.0000x reference)
import functools

import jax
import jax.numpy as jnp
from jax import lax
from jax.experimental import pallas as pl
from jax.experimental.pallas import tpu as pltpu
from jax.experimental.pallas import tpu_sc as plsc

_VOCAB = 1000
_LEN = 50
_DPOS = 64
_DOUT = _VOCAB + _DPOS
_NC = 2
_NS = 16
_NW = _NC * _NS
_LANES = 16
_SLAB = 8
_SPL = _DOUT // _SLAB
_NSLAB = _LEN * _SPL
_NBUF = 4


def _sc_body(xt_hbm, pos_hbm, out_hbm, xt_v, pos_v, slab_v, st_v, sems):
    batch = xt_hbm.shape[1]
    nchunk = batch // _LANES
    wid = lax.axis_index("s") * _NC + lax.axis_index("c")

    pltpu.sync_copy(xt_hbm, xt_v)
    pltpu.sync_copy(pos_hbm, pos_v)

    lane = lax.broadcasted_iota(jnp.int32, (_LANES,), 0)
    ones = jnp.full((_LANES,), 1.0, jnp.float32)
    zeros = jnp.zeros((_LANES,), jnp.float32)

    g0 = (wid * _NSLAB) // _NW
    g1 = ((wid + 1) * _NSLAB) // _NW

    def zero_body(i, carry):
        p = i // (_SLAB * nchunk)
        r = lax.rem(i // nchunk, _SLAB)
        k = lax.rem(i, nchunk)
        slab_v[p, r, pl.ds(k * _LANES, _LANES)] = zeros
        return carry
    lax.fori_loop(0, _NBUF * _SLAB * nchunk, zero_body, 0)

    def scatter_tokens(buf, l, vlo, val):
        def chunk_body(c, carry):
            xs = xt_v[l, pl.ds(c * _LANES, _LANES)]
            rel = xs - vlo
            mask = (rel >= 0) & (rel < _SLAB)
            cols = lane + c * _LANES
            plsc.store_scatter(buf, [rel, cols], val, mask=mask)
            return carry
        lax.fori_loop(0, nchunk, chunk_body, 0)

    def fill_pos_rows(buf, l, vlo, clear):
        def row_body(r, carry):
            v = vlo + r
            @pl.when(v >= _VOCAB)
            def _():
                j = v - _VOCAB
                c = lax.shift_right_logical(j, 4)
                chunk = pos_v[l, pl.ds(c * _LANES, _LANES)]
                sel = lane == (j - c * _LANES)
                val = jnp.sum(jnp.where(sel, chunk, 0.0))
                val = lax.select(clear, jnp.float32(0.0), val)
                vec = jnp.full((_LANES,), 0.0, jnp.float32) + val
                def col_body(k, carry2):
                    buf[r, pl.ds(k * _LANES, _LANES)] = vec
                    return carry2
                lax.fori_loop(0, nchunk, col_body, 0)
            return carry
        lax.fori_loop(0, _SLAB, row_body, 0)

    def issue(g):
        p = lax.rem(g, _NBUF)
        l = g // _SPL
        vlo = pl.multiple_of(lax.rem(g, _SPL) * _SLAB, _SLAB)
        buf = slab_v.at[p]
        fill_pos_rows(buf, l, vlo, False)
        scatter_tokens(buf, l, vlo, ones)
        st_v[p, 0] = l
        st_v[p, 1] = vlo
        pltpu.async_copy(buf, out_hbm.at[l, pl.ds(vlo, _SLAB)], sems.at[p])

    def drain(g):
        p = lax.rem(g, _NBUF)
        l = st_v[p, 0]
        vlo = pl.multiple_of(st_v[p, 1], _SLAB)
        buf = slab_v.at[p]
        pltpu.make_async_copy(buf, out_hbm.at[l, pl.ds(vlo, _SLAB)],
                              sems.at[p]).wait()
        fill_pos_rows(buf, l, vlo, True)
        scatter_tokens(buf, l, vlo, zeros)

    nslab_w = g1 - g0

    def prologue(k, carry):
        @pl.when(k < nslab_w)
        def _():
            issue(g0 + k)
        return carry
    lax.fori_loop(0, _NBUF, prologue, 0)

    def step(g, carry):
        drain(g)
        @pl.when(g + _NBUF < g1)
        def _():
            issue(g + _NBUF)
        return carry
    lax.fori_loop(g0, g1, step, 0)


def kernel(X, pos_emb):
    batch = X.shape[0]
    xt = jnp.transpose(X.astype(jnp.int32))
    pos = pos_emb.astype(jnp.float32)
    run = pl.kernel(
        _sc_body,
        out_type=jax.ShapeDtypeStruct((_LEN, _DOUT, batch), jnp.float32),
        mesh=plsc.VectorSubcoreMesh(core_axis_name="c", subcore_axis_name="s",
                                    num_cores=_NC, num_subcores=_NS),
        compiler_params=pltpu.CompilerParams(needs_layout_passes=False),
        scratch_types=[
            pltpu.VMEM((_LEN, batch), jnp.int32),
            pltpu.VMEM((_LEN, _DPOS), jnp.float32),
            pltpu.VMEM((_NBUF, _SLAB, batch), jnp.float32),
            pltpu.SMEM((_NBUF, 2), jnp.int32),
            pltpu.SemaphoreType.DMA((_NBUF,)),
        ],
    )
    out_t = run(xt, pos)
    return jnp.transpose(out_t, (2, 0, 1))


# speedup vs baseline: 2.7439x; 2.7439x over previous
import functools

import jax
import jax.numpy as jnp
from jax import lax
from jax.experimental import pallas as pl
from jax.experimental.pallas import tpu as pltpu
from jax.experimental.pallas import tpu_sc as plsc

_VOCAB = 1000
_LEN = 50
_DPOS = 64
_DOUT = _VOCAB + _DPOS
_NC = 2
_NS = 16
_NW = _NC * _NS
_LANES = 16
_SLAB = 56
_SPL = _DOUT // _SLAB
_NSLAB = _LEN * _SPL
_NBUF = 2
_NROWS = 3


def _sc_body(xt_hbm, pos_hbm, out_hbm, xt_v, pos_v, slab_v, st_v, sems):
    batch = xt_hbm.shape[0] // _LEN
    nchunk = batch // _LANES
    wid = lax.axis_index("s") * _NC + lax.axis_index("c")

    g0 = (wid * _NSLAB) // _NW
    g1 = ((wid + 1) * _NSLAB) // _NW
    lbase = jnp.minimum(g0 // _SPL, _LEN - _NROWS)

    pltpu.sync_copy(
        xt_hbm.at[pl.ds(pl.multiple_of(lbase * batch, batch),
                        _NROWS * batch)], xt_v)
    pltpu.sync_copy(
        pos_hbm.at[pl.ds(pl.multiple_of(lbase * _DPOS, _DPOS),
                         _NROWS * _DPOS)], pos_v)

    lane = lax.broadcasted_iota(jnp.int32, (_LANES,), 0)
    ones = jnp.full((_LANES,), 1.0, jnp.float32)
    zeros = jnp.zeros((_LANES,), jnp.float32)

    def zero_body(i, carry):
        p = i // (_SLAB * nchunk)
        r = lax.rem(i // nchunk, _SLAB)
        k = lax.rem(i, nchunk)
        slab_v[p, r, pl.ds(k * _LANES, _LANES)] = zeros
        return carry
    lax.fori_loop(0, _NBUF * _SLAB * nchunk, zero_body, 0)

    def scatter_tokens(buf, lrel, vlo, val):
        for c in range(nchunk):
            xs = xt_v[pl.ds(lrel * batch + c * _LANES, _LANES)]
            rel = xs - vlo
            mask = (rel >= 0) & (rel < _SLAB)
            cols = lane + c * _LANES
            plsc.store_scatter(buf, [rel, cols], val, mask=mask)

    def fill_pos_rows(buf, lrel, vlo, clear):
        r_start = jnp.clip(_VOCAB - vlo, 0, _SLAB)

        def row_body(r, carry):
            j = (vlo + r) - _VOCAB
            c = lax.shift_right_logical(j, 4)
            chunk = pos_v[pl.ds(lrel * _DPOS + c * _LANES, _LANES)]
            sel = lane == (j - c * _LANES)
            val = jnp.sum(jnp.where(sel, chunk, 0.0))
            val = lax.select(clear, jnp.float32(0.0), val)
            vec = jnp.full((_LANES,), 0.0, jnp.float32) + val
            for k in range(nchunk):
                buf[r, pl.ds(k * _LANES, _LANES)] = vec
            return carry
        lax.fori_loop(r_start, _SLAB, row_body, 0)

    def issue(g):
        p = lax.rem(g, _NBUF)
        l = g // _SPL
        vlo = pl.multiple_of(lax.rem(g, _SPL) * _SLAB, _SLAB)
        buf = slab_v.at[p]
        fill_pos_rows(buf, l - lbase, vlo, False)
        scatter_tokens(buf, l - lbase, vlo, ones)
        st_v[p, 0] = l
        st_v[p, 1] = vlo
        pltpu.async_copy(buf, out_hbm.at[l, pl.ds(vlo, _SLAB)], sems.at[p])

    def drain(g):
        p = lax.rem(g, _NBUF)
        l = st_v[p, 0]
        vlo = pl.multiple_of(st_v[p, 1], _SLAB)
        buf = slab_v.at[p]
        pltpu.make_async_copy(buf, out_hbm.at[l, pl.ds(vlo, _SLAB)],
                              sems.at[p]).wait()
        fill_pos_rows(buf, l - lbase, vlo, True)
        scatter_tokens(buf, l - lbase, vlo, zeros)

    def prologue(k, carry):
        @pl.when(g0 + k < g1)
        def _():
            issue(g0 + k)
        return carry
    lax.fori_loop(0, _NBUF, prologue, 0)

    def step(g, carry):
        drain(g)
        @pl.when(g + _NBUF < g1)
        def _():
            issue(g + _NBUF)
        return carry
    lax.fori_loop(g0, g1, step, 0)


def kernel(X, pos_emb):
    batch = X.shape[0]
    xt = jnp.transpose(X.astype(jnp.int32)).reshape(-1)
    pos = pos_emb.astype(jnp.float32).reshape(-1)
    run = pl.kernel(
        _sc_body,
        out_type=jax.ShapeDtypeStruct((_LEN, _DOUT, batch), jnp.float32),
        mesh=plsc.VectorSubcoreMesh(core_axis_name="c", subcore_axis_name="s",
                                    num_cores=_NC, num_subcores=_NS),
        compiler_params=pltpu.CompilerParams(needs_layout_passes=False),
        scratch_types=[
            pltpu.VMEM((_NROWS * batch,), jnp.int32),
            pltpu.VMEM((_NROWS * _DPOS,), jnp.float32),
            pltpu.VMEM((_NBUF, _SLAB, batch), jnp.float32),
            pltpu.SMEM((_NBUF, 2), jnp.int32),
            pltpu.SemaphoreType.DMA((_NBUF,)),
        ],
    )
    out_t = run(xt, pos)
    return jnp.transpose(out_t, (2, 0, 1))


# speedup vs baseline: 3.5106x; 1.2794x over previous
import functools

import jax
import jax.numpy as jnp
from jax import lax
from jax.experimental import pallas as pl
from jax.experimental.pallas import tpu as pltpu
from jax.experimental.pallas import tpu_sc as plsc

_VOCAB = 1000
_LEN = 50
_DPOS = 64
_DOUT = _VOCAB + _DPOS
_NC = 2
_NS = 16
_NW = _NC * _NS
_LANES = 16
_SLAB = 56
_SPL = _DOUT // _SLAB
_NSLAB = _LEN * _SPL
_NBUF = 2
_NROWS = 3


def _sc_body(xt_hbm, pos_hbm, out_hbm, xt_v, pos_v, slab_v, st_v, sems):
    batch = xt_hbm.shape[0] // _LEN
    nchunk = batch // _LANES
    wid = lax.axis_index("s") * _NC + lax.axis_index("c")

    g0 = (wid * _NSLAB) // _NW
    g1 = ((wid + 1) * _NSLAB) // _NW
    lbase = jnp.minimum(g0 // _SPL, _LEN - _NROWS)

    pltpu.sync_copy(
        xt_hbm.at[pl.ds(pl.multiple_of(lbase * batch, batch),
                        _NROWS * batch)], xt_v)
    pltpu.sync_copy(
        pos_hbm.at[pl.ds(pl.multiple_of(lbase * _DPOS, _DPOS),
                         _NROWS * _DPOS)], pos_v)

    lane = lax.broadcasted_iota(jnp.int32, (_LANES,), 0)
    ones = jnp.full((_LANES,), 1.0, jnp.float32)
    zeros = jnp.zeros((_LANES,), jnp.float32)

    for p in range(_NBUF):
        def zero_row(r, carry):
            for k in range(nchunk):
                slab_v[p, r, pl.ds(k * _LANES, _LANES)] = zeros
            return carry
        lax.fori_loop(0, _SLAB, zero_row, 0)

    def scatter_tokens(buf, lrel, vlo, val):
        for c in range(nchunk):
            xs = xt_v[pl.ds(lrel * batch + c * _LANES, _LANES)]
            rel = xs - vlo
            mask = (rel >= 0) & (rel < _SLAB)
            cols = lane + c * _LANES
            plsc.store_scatter(buf, [rel, cols], val, mask=mask)

    def fill_pos_rows(buf, lrel, vlo, clear):
        r_start = jnp.clip(_VOCAB - vlo, 0, _SLAB)

        def row_body(r, carry):
            j = (vlo + r) - _VOCAB
            c = lax.shift_right_logical(j, 4)
            chunk = pos_v[pl.ds(lrel * _DPOS + c * _LANES, _LANES)]
            sel = lane == (j - c * _LANES)
            val = jnp.sum(jnp.where(sel, chunk, 0.0))
            val = lax.select(clear, jnp.float32(0.0), val)
            vec = jnp.full((_LANES,), 0.0, jnp.float32) + val
            for k in range(nchunk):
                buf[r, pl.ds(k * _LANES, _LANES)] = vec
            return carry
        lax.fori_loop(r_start, _SLAB, row_body, 0)

    def issue(g):
        p = lax.rem(g, _NBUF)
        l = g // _SPL
        vlo = pl.multiple_of(lax.rem(g, _SPL) * _SLAB, _SLAB)
        buf = slab_v.at[p]
        fill_pos_rows(buf, l - lbase, vlo, False)
        scatter_tokens(buf, l - lbase, vlo, ones)
        st_v[p, 0] = l
        st_v[p, 1] = vlo
        pltpu.async_copy(buf, out_hbm.at[l, pl.ds(vlo, _SLAB)], sems.at[p])

    def drain(g):
        p = lax.rem(g, _NBUF)
        l = st_v[p, 0]
        vlo = pl.multiple_of(st_v[p, 1], _SLAB)
        buf = slab_v.at[p]
        pltpu.make_async_copy(buf, out_hbm.at[l, pl.ds(vlo, _SLAB)],
                              sems.at[p]).wait()
        fill_pos_rows(buf, l - lbase, vlo, True)
        scatter_tokens(buf, l - lbase, vlo, zeros)

    def prologue(k, carry):
        @pl.when(g0 + k < g1)
        def _():
            issue(g0 + k)
        return carry
    lax.fori_loop(0, _NBUF, prologue, 0)

    def step(g, carry):
        drain(g)
        @pl.when(g + _NBUF < g1)
        def _():
            issue(g + _NBUF)
        return carry
    lax.fori_loop(g0, g1, step, 0)


def kernel(X, pos_emb):
    batch = X.shape[0]
    xt = jnp.transpose(X.astype(jnp.int32)).reshape(-1)
    pos = pos_emb.astype(jnp.float32).reshape(-1)
    run = pl.kernel(
        _sc_body,
        out_type=jax.ShapeDtypeStruct((_LEN, _DOUT, batch), jnp.float32),
        mesh=plsc.VectorSubcoreMesh(core_axis_name="c", subcore_axis_name="s",
                                    num_cores=_NC, num_subcores=_NS),
        compiler_params=pltpu.CompilerParams(needs_layout_passes=False),
        scratch_types=[
            pltpu.VMEM((_NROWS * batch,), jnp.int32),
            pltpu.VMEM((_NROWS * _DPOS,), jnp.float32),
            pltpu.VMEM((_NBUF, _SLAB, batch), jnp.float32),
            pltpu.SMEM((_NBUF, 2), jnp.int32),
            pltpu.SemaphoreType.DMA((_NBUF,)),
        ],
    )
    out_t = run(xt, pos)
    return jnp.transpose(out_t, (2, 0, 1))
